# Initial kernel scaffold; baseline (speedup 1.0000x reference)
import jax
import jax.numpy as jnp
from jax.experimental import pallas as pl


def kernel(x_user, x_item, edge_index_ui, edge_index_iu, W_enc_u, W_enc_i, Wl_ui, bl_ui, Wr_ui, Wl_iu, bl_iu, Wr_iu, Wh, bh):
    raise NotImplementedError("write your pallas kernel here")


# speedup vs baseline: 1.3277x; 1.3277x over previous
import functools

import jax
import jax.numpy as jnp
from jax import lax
from jax.experimental import pallas as pl
from jax.experimental.pallas import tpu as pltpu
from jax.experimental.pallas import tpu_sc as plsc

N = 10000
E = 160000
H = 256
HH = 128
NS = 16
CH = 128
NCH = 80
NB = 4
EPT = NCH * CH
EP = NS * EPT
OUTR = 10240
PASS_ROWS = 5120
ACC2 = 5632
DUMW = 512
ZCH = (128, 128, 96)
WB = PASS_ROWS // NS
WBC = 5
WBR = WB // WBC

BLK = 1000
GRID = N // BLK

_f32 = jnp.float32


def _dotT(a, b):
    return lax.dot_general(a, b, (((1,), (1,)), ((), ())),
                           preferred_element_type=_f32)


def _sc_body_factory(two_phase, with_counts):
    n_tab = 4 if two_phase else 2
    n_rel = 2 if two_phase else 1
    n_edge = n_rel * 3

    def body(*refs):
        i = 0
        tabs = refs[i:i + n_tab]; i += n_tab
        edges = refs[i:i + n_edge]; i += n_edge
        z2 = refs[i]; i += 1
        if with_counts:
            o2 = refs[i]; i += 1
        outs = refs[i:i + n_tab]; i += n_tab
        if with_counts:
            cntA = refs[i]; cntB = refs[i + 1]; i += 2
        src_v, dst_v, b0, b1, b2, b3, acc = refs[i:i + 7]
        sems = refs[i + 7:i + 11]
        bufs = (b0, b1, b2, b3)

        cid = lax.axis_index("c")
        sid = lax.axis_index("s")

        def one_pass(table, srcE, dstE, out, p):
            if table is None:
                pltpu.sync_copy(o2, b1)
            else:
                pltpu.sync_copy(srcE.at[sid], src_v)
            pltpu.sync_copy(dstE.at[sid], dst_v)
            pltpu.sync_copy(z2, b0)
            zoff = 0
            for zn in ZCH:
                pltpu.sync_copy(b0.at[pl.ds(0, zn)],
                                acc.at[pl.ds((ACC2 // NS) * sid + zoff, zn)])
                zoff += zn
            plsc.subcore_barrier()

            if table is None:
                def outer_c(g, carry):
                    pltpu.sync_copy(b1, acc.at[dst_v.at[g]], add=True)
                    return carry

                lax.fori_loop(0, NCH, outer_c, 0)
            else:
                for b in range(NB):
                    pltpu.async_copy(table.at[src_v.at[b]], bufs[b], sems[b])

                def outer(g, carry):
                    base = g * NB
                    for b in range(NB):
                        j = base + b
                        pltpu.make_async_copy(
                            table.at[src_v.at[j]], bufs[b], sems[b]).wait()
                        pltpu.sync_copy(bufs[b], acc.at[dst_v.at[j]], add=True)

                        @pl.when(j + NB < NCH)
                        def _():
                            pltpu.async_copy(
                                table.at[src_v.at[j + NB]], bufs[b], sems[b])
                    return carry

                lax.fori_loop(0, NCH // NB, outer, 0)
            plsc.subcore_barrier()

            for k in range(WBC):
                off = WB * sid + WBR * k
                pltpu.sync_copy(acc.at[pl.ds(off, WBR)], b0.at[pl.ds(0, WBR)])
                pltpu.sync_copy(b0.at[pl.ds(0, WBR)],
                                out.at[pl.ds(PASS_ROWS * p + off, WBR)])
            plsc.subcore_barrier()

        def phase(table, erefs, out):
            srcE, dLo, dHi = erefs
            one_pass(table, srcE, dLo, out, 0)
            one_pass(table, srcE, dHi, out, 1)

        eA = edges[0:3]
        eB = edges[3:6] if two_phase else None

        @pl.when(cid == 0)
        def _():
            phase(tabs[0], eA, outs[0])
            if two_phase:
                phase(tabs[2], eB, outs[2])
            if with_counts:
                phase(None, eA, cntA)

        @pl.when(cid == 1)
        def _():
            phase(tabs[1], eA, outs[1])
            if two_phase:
                phase(tabs[3], eB, outs[3])
            if with_counts:
                phase(None, eB if two_phase else eA, cntB)

    return body


def _make_sc_kernel(two_phase, with_counts):
    n_tab = 4 if two_phase else 2
    out_type = [jax.ShapeDtypeStruct((OUTR, HH), _f32)
                for _ in range(n_tab + (2 if with_counts else 0))]
    scratch = [
        pltpu.VMEM((NCH, CH), jnp.int32),
        pltpu.VMEM((NCH, CH), jnp.int32),
        pltpu.VMEM((CH, HH), _f32),
        pltpu.VMEM((CH, HH), _f32),
        pltpu.VMEM((CH, HH), _f32),
        pltpu.VMEM((CH, HH), _f32),
        pltpu.VMEM_SHARED((ACC2, HH), _f32),
        pltpu.SemaphoreType.DMA,
        pltpu.SemaphoreType.DMA,
        pltpu.SemaphoreType.DMA,
        pltpu.SemaphoreType.DMA,
    ]
    return pl.kernel(
        _sc_body_factory(two_phase, with_counts),
        out_type=out_type,
        mesh=plsc.VectorSubcoreMesh(core_axis_name="c", subcore_axis_name="s"),
        scratch_types=scratch,
    )


_sc_agg2_cnt = _make_sc_kernel(True, True)
_sc_agg2 = _make_sc_kernel(True, False)
_sc_agg1 = _make_sc_kernel(False, False)


def _relu(x):
    return jnp.maximum(x, 0.0)


def _mm_half(x0, x1, w, h):
    return (_dotT(x0, w[h * HH:(h + 1) * HH, 0:HH]) +
            _dotT(x1, w[h * HH:(h + 1) * HH, HH:H]))


def _ka_body(xu_ref, xi_ref, weu_ref, wei_ref, wlu_ref, wru_ref, wli_ref,
             wri_ref, yu0_ref, yu1_ref, yi0_ref, yi1_ref, ru0_ref, ru1_ref,
             ri0_ref, ri1_ref):
    xu = xu_ref[...]
    xi = xi_ref[...]
    weu = weu_ref[...]
    wei = wei_ref[...]
    hu0 = _relu(_dotT(xu, weu[0:HH]))
    hu1 = _relu(_dotT(xu, weu[HH:H]))
    hi0 = _relu(_dotT(xi, wei[0:HH]))
    hi1 = _relu(_dotT(xi, wei[HH:H]))
    wlu = wlu_ref[...]
    wru = wru_ref[...]
    wli = wli_ref[...]
    wri = wri_ref[...]
    yu0_ref[...] = _mm_half(hu0, hu1, wlu, 0)
    yu1_ref[...] = _mm_half(hu0, hu1, wlu, 1)
    ru0_ref[...] = _mm_half(hu0, hu1, wru, 0)
    ru1_ref[...] = _mm_half(hu0, hu1, wru, 1)
    yi0_ref[...] = _mm_half(hi0, hi1, wli, 0)
    yi1_ref[...] = _mm_half(hi0, hi1, wli, 1)
    ri0_ref[...] = _mm_half(hi0, hi1, wri, 0)
    ri1_ref[...] = _mm_half(hi0, hi1, wri, 1)


def _combine(a0, a1, c, b, r0, r1):
    rc = 1.0 / jnp.maximum(c, 1.0)
    x0 = _relu(a0 * rc + b[:, 0:HH] + r0)
    x1 = _relu(a1 * rc + b[:, HH:H] + r1)
    return x0, x1


def _kb_body(full, ai0_ref, ai1_ref, au0_ref, au1_ref, ri0_ref, ri1_ref,
             ru0_ref, ru1_ref, ci_ref, cu_ref, bui_ref, biu_ref,
             wlu_ref, wru_ref, wli_ref, wri_ref, *out_refs):
    xi0, xi1 = _combine(ai0_ref[...], ai1_ref[...], ci_ref[...], bui_ref[...],
                        ri0_ref[...], ri1_ref[...])
    xu0, xu1 = _combine(au0_ref[...], au1_ref[...], cu_ref[...], biu_ref[...],
                        ru0_ref[...], ru1_ref[...])
    wli = wli_ref[...]
    wri = wri_ref[...]
    if full:
        (yu0_ref, yu1_ref, yi0_ref, yi1_ref, ru0o_ref, ru1o_ref, ri0o_ref,
         ri1o_ref) = out_refs
        wlu = wlu_ref[...]
        wru = wru_ref[...]
        yu0_ref[...] = _mm_half(xu0, xu1, wlu, 0)
        yu1_ref[...] = _mm_half(xu0, xu1, wlu, 1)
        ru0o_ref[...] = _mm_half(xu0, xu1, wru, 0)
        ru1o_ref[...] = _mm_half(xu0, xu1, wru, 1)
        ri0o_ref[...] = _mm_half(xi0, xi1, wri, 0)
        ri1o_ref[...] = _mm_half(xi0, xi1, wri, 1)
    else:
        yi0_ref, yi1_ref, ru0o_ref, ru1o_ref = out_refs
        wru = wru_ref[...]
        ru0o_ref[...] = _mm_half(xu0, xu1, wru, 0)
        ru1o_ref[...] = _mm_half(xu0, xu1, wru, 1)
    yi0_ref[...] = _mm_half(xi0, xi1, wli, 0)
    yi1_ref[...] = _mm_half(xi0, xi1, wli, 1)


def _kc_body(au0_ref, au1_ref, ru0_ref, ru1_ref, cu_ref, biu_ref, wh_ref,
             bh_ref, out_ref):
    xu0, xu1 = _combine(au0_ref[...], au1_ref[...], cu_ref[...], biu_ref[...],
                        ru0_ref[...], ru1_ref[...])
    wh = wh_ref[...]
    out_ref[...] = (_dotT(xu0, wh[:, 0:HH]) + _dotT(xu1, wh[:, HH:H])
                    + bh_ref[...])


def _rows(c):
    return pl.BlockSpec((BLK, c), lambda i: (i, 0))


def _whole(shape):
    return pl.BlockSpec(shape, lambda i: tuple(0 for _ in shape))


_half_out = [jax.ShapeDtypeStruct((N, HH), _f32)]


def _ka_call(xu, xi, weu, wei, wlu, wru, wli, wri):
    return pl.pallas_call(
        _ka_body,
        grid=(GRID,),
        in_specs=[_rows(H), _rows(H)] + [_whole((H, H))] * 6,
        out_specs=[_rows(HH)] * 8,
        out_shape=_half_out * 8,
    )(xu, xi, weu, wei, wlu, wru, wli, wri)


def _kb_call(full, ai0, ai1, au0, au1, ri0, ri1, ru0, ru1, ci, cu, bui, biu,
             wlu, wru, wli, wri):
    n_out = 8 if full else 4
    return pl.pallas_call(
        functools.partial(_kb_body, full),
        grid=(GRID,),
        in_specs=[_rows(HH)] * 10 + [_whole((1, H))] * 2
                 + [_whole((H, H))] * 4,
        out_specs=[_rows(HH)] * n_out,
        out_shape=_half_out * n_out,
    )(ai0, ai1, au0, au1, ri0, ri1, ru0, ru1, ci, cu, bui, biu,
      wlu, wru, wli, wri)


def _kc_call(au0, au1, ru0, ru1, cu, biu, wh, bh):
    d_out = wh.shape[0]
    return pl.pallas_call(
        _kc_body,
        grid=(GRID,),
        in_specs=[_rows(HH)] * 5 + [_whole((1, H)),
                                    _whole((d_out, H)), _whole((1, d_out))],
        out_specs=_rows(d_out),
        out_shape=jax.ShapeDtypeStruct((N, d_out), _f32),
    )(au0, au1, ru0, ru1, cu, biu, wh, bh)


def _prep_edges(edge_index):
    src = edge_index[0].astype(jnp.int32)
    dst = edge_index[1].astype(jnp.int32)
    pad = EP - E
    src = jnp.concatenate([src, jnp.zeros((pad,), jnp.int32)])
    dst = jnp.concatenate([dst, jnp.full((pad,), N, jnp.int32)])
    dummy = PASS_ROWS + (jnp.arange(EP, dtype=jnp.int32) % DUMW)
    d_lo = jnp.where(dst < PASS_ROWS, dst, dummy)
    d_hi = jnp.where((dst >= PASS_ROWS) & (dst < N), dst - PASS_ROWS, dummy)
    return (src.reshape(NS, NCH, CH), d_lo.reshape(NS, NCH, CH),
            d_hi.reshape(NS, NCH, CH))


def kernel(x_user, x_item, edge_index_ui, edge_index_iu, W_enc_u, W_enc_i,
           Wl_ui, bl_ui, Wr_ui, Wl_iu, bl_iu, Wr_iu, Wh, bh):
    src_ui, dlo_ui, dhi_ui = _prep_edges(edge_index_ui)
    src_iu, dlo_iu, dhi_iu = _prep_edges(edge_index_iu)
    z2 = jnp.zeros((CH, HH), _f32)
    o2 = jnp.ones((CH, HH), _f32)

    bui = [bl_ui[l].reshape(1, H) for l in range(3)]
    biu = [bl_iu[l].reshape(1, H) for l in range(3)]
    bh2 = bh.reshape(1, -1)

    yu0, yu1, yi0, yi1, ru0, ru1, ri0, ri1 = _ka_call(
        x_user, x_item, W_enc_u, W_enc_i, Wl_ui[0], Wr_iu[0], Wl_iu[0],
        Wr_ui[0])

    ai0, ai1, au0, au1, ci, cu = _sc_agg2_cnt(
        yu0, yu1, yi0, yi1, src_ui, dlo_ui, dhi_ui,
        src_iu, dlo_iu, dhi_iu, z2, o2)

    yu0, yu1, yi0, yi1, ru0, ru1, ri0, ri1 = _kb_call(
        True, ai0, ai1, au0, au1, ri0, ri1, ru0, ru1, ci, cu, bui[0], biu[0],
        Wl_ui[1], Wr_iu[1], Wl_iu[1], Wr_ui[1])

    ai0, ai1, au0, au1 = _sc_agg2(
        yu0, yu1, yi0, yi1, src_ui, dlo_ui, dhi_ui, src_iu, dlo_iu, dhi_iu,
        z2)

    yi0, yi1, ru0, ru1 = _kb_call(
        False, ai0, ai1, au0, au1, ri0, ri1, ru0, ru1, ci, cu, bui[1], biu[1],
        Wl_ui[2], Wr_iu[2], Wl_iu[2], Wr_ui[2])

    au0, au1 = _sc_agg1(yi0, yi1, src_iu, dlo_iu, dhi_iu, z2)

    return _kc_call(au0, au1, ru0, ru1, cu, biu[2], Wh, bh2)


# speedup vs baseline: 2.1073x; 1.5872x over previous
import functools

import jax
import jax.numpy as jnp
from jax import lax
from jax.experimental import pallas as pl
from jax.experimental.pallas import tpu as pltpu
from jax.experimental.pallas import tpu_sc as plsc

N = 10000
E = 160000
H = 256
HH = 128
HQ = 64
NS = 16
CH = 128
NCH = 80
NB = 4
EPT = NCH * CH
EP = NS * EPT
ACC = 10240
WB = ACC // NS
WBC = 5
WBR = WB // WBC

BLK = 1000
GRID = N // BLK

_f32 = jnp.float32


def _dotT(a, b):
    return lax.dot_general(a, b, (((1,), (1,)), ((), ())),
                           preferred_element_type=_f32)


def _sc_body_factory(two_phase, with_counts):
    n_rel = 2 if two_phase else 1
    n_tab = 4 * n_rel
    n_edge = 2 * n_rel

    def body(*refs):
        i = 0
        tabs = refs[i:i + n_tab]; i += n_tab
        edges = refs[i:i + n_edge]; i += n_edge
        z2 = refs[i]; i += 1
        if with_counts:
            o2 = refs[i]; i += 1
        outs = refs[i:i + n_tab]; i += n_tab
        if with_counts:
            cnts = refs[i:i + 2]; i += 2
        src_v, dst_v, b0, b1, b2, b3, acc = refs[i:i + 7]
        sems = refs[i + 7:i + 11]
        bufs = (b0, b1, b2, b3)

        cid = lax.axis_index("c")
        sid = lax.axis_index("s")

        def phase(table, srcE, dstE, out):
            if table is None:
                pltpu.sync_copy(o2, b1)
            else:
                pltpu.sync_copy(srcE.at[sid], src_v)
            pltpu.sync_copy(dstE.at[sid], dst_v)
            pltpu.sync_copy(z2, b0)
            for k in range(WB // WBR):
                pltpu.sync_copy(b0, acc.at[pl.ds(WB * sid + WBR * k, WBR)])
            plsc.subcore_barrier()

            if table is None:
                def outer_c(g, carry):
                    pltpu.sync_copy(b1, acc.at[dst_v.at[g]], add=True)
                    return carry

                lax.fori_loop(0, NCH, outer_c, 0)
            else:
                for b in range(NB):
                    pltpu.async_copy(table.at[src_v.at[b]], bufs[b], sems[b])

                def outer(g, carry):
                    base = g * NB
                    for b in range(NB):
                        j = base + b
                        pltpu.make_async_copy(
                            table.at[src_v.at[j]], bufs[b], sems[b]).wait()
                        pltpu.sync_copy(bufs[b], acc.at[dst_v.at[j]], add=True)

                        @pl.when(j + NB < NCH)
                        def _():
                            pltpu.async_copy(
                                table.at[src_v.at[j + NB]], bufs[b], sems[b])
                    return carry

                lax.fori_loop(0, NCH // NB, outer, 0)
            plsc.subcore_barrier()

            for k in range(WBC):
                off = WB * sid + WBR * k
                pltpu.sync_copy(acc.at[pl.ds(off, WBR)], b0)
                pltpu.sync_copy(b0, out.at[pl.ds(off, WBR)])
            plsc.subcore_barrier()

        eA = edges[0:2]
        eB = edges[2:4] if two_phase else None

        def run(q0):
            for q in (q0, q0 + 1):
                phase(tabs[q], eA[0], eA[1], outs[q])
                if two_phase:
                    phase(tabs[4 + q], eB[0], eB[1], outs[4 + q])

        @pl.when(cid == 0)
        def _():
            run(0)
            if with_counts:
                phase(None, eA[0], eA[1], cnts[0])

        @pl.when(cid == 1)
        def _():
            run(2)
            if with_counts:
                e = eB if two_phase else eA
                phase(None, e[0], e[1], cnts[1])

    return body


def _make_sc_kernel(two_phase, with_counts):
    n_tab = 4 * (2 if two_phase else 1)
    out_type = [jax.ShapeDtypeStruct((ACC, HQ), _f32)
                for _ in range(n_tab + (2 if with_counts else 0))]
    scratch = [
        pltpu.VMEM((NCH, CH), jnp.int32),
        pltpu.VMEM((NCH, CH), jnp.int32),
        pltpu.VMEM((WBR, HQ), _f32),
        pltpu.VMEM((WBR, HQ), _f32),
        pltpu.VMEM((WBR, HQ), _f32),
        pltpu.VMEM((WBR, HQ), _f32),
        pltpu.VMEM_SHARED((ACC, HQ), _f32),
        pltpu.SemaphoreType.DMA,
        pltpu.SemaphoreType.DMA,
        pltpu.SemaphoreType.DMA,
        pltpu.SemaphoreType.DMA,
    ]
    return pl.kernel(
        _sc_body_factory(two_phase, with_counts),
        out_type=out_type,
        mesh=plsc.VectorSubcoreMesh(core_axis_name="c", subcore_axis_name="s"),
        scratch_types=scratch,
        compiler_params=pltpu.CompilerParams(use_tc_tiling_on_sc=False),
    )


_sc_agg2_cnt = _make_sc_kernel(True, True)
_sc_agg2 = _make_sc_kernel(True, False)
_sc_agg1 = _make_sc_kernel(False, False)


def _relu(x):
    return jnp.maximum(x, 0.0)


def _mm_half(x0, x1, w, h):
    return (_dotT(x0, w[h * HH:(h + 1) * HH, 0:HH]) +
            _dotT(x1, w[h * HH:(h + 1) * HH, HH:H]))


def _store_quarters(refs, h0, h1):
    refs[0][...] = h0[:, 0:HQ]
    refs[1][...] = h0[:, HQ:HH]
    refs[2][...] = h1[:, 0:HQ]
    refs[3][...] = h1[:, HQ:HH]


def _ka_body(xu_ref, xi_ref, weu_ref, wei_ref, wlu_ref, wru_ref, wli_ref,
             wri_ref, *out_refs):
    xu = xu_ref[...]
    xi = xi_ref[...]
    weu = weu_ref[...]
    wei = wei_ref[...]
    hu0 = _relu(_dotT(xu, weu[0:HH]))
    hu1 = _relu(_dotT(xu, weu[HH:H]))
    hi0 = _relu(_dotT(xi, wei[0:HH]))
    hi1 = _relu(_dotT(xi, wei[HH:H]))
    wlu = wlu_ref[...]
    wru = wru_ref[...]
    wli = wli_ref[...]
    wri = wri_ref[...]
    _store_quarters(out_refs[0:4], _mm_half(hu0, hu1, wlu, 0),
                    _mm_half(hu0, hu1, wlu, 1))
    _store_quarters(out_refs[4:8], _mm_half(hi0, hi1, wli, 0),
                    _mm_half(hi0, hi1, wli, 1))
    out_refs[8][...] = _mm_half(hu0, hu1, wru, 0)
    out_refs[9][...] = _mm_half(hu0, hu1, wru, 1)
    out_refs[10][...] = _mm_half(hi0, hi1, wri, 0)
    out_refs[11][...] = _mm_half(hi0, hi1, wri, 1)


def _combine(aq, c, b, r0, r1):
    rc = 1.0 / jnp.maximum(c, 1.0)
    x0 = _relu(jnp.concatenate([aq[0] * rc + b[:, 0:HQ],
                                aq[1] * rc + b[:, HQ:HH]], axis=1) + r0)
    x1 = _relu(jnp.concatenate([aq[2] * rc + b[:, HH:HH + HQ],
                                aq[3] * rc + b[:, HH + HQ:H]], axis=1) + r1)
    return x0, x1


def _kb_body(full, ai0_ref, ai1_ref, ai2_ref, ai3_ref, au0_ref, au1_ref,
             au2_ref, au3_ref, ri0_ref, ri1_ref, ru0_ref, ru1_ref, ci_ref,
             cu_ref, bui_ref, biu_ref, wlu_ref, wru_ref, wli_ref, wri_ref,
             *out_refs):
    xi0, xi1 = _combine([ai0_ref[...], ai1_ref[...], ai2_ref[...],
                         ai3_ref[...]], ci_ref[...], bui_ref[...],
                        ri0_ref[...], ri1_ref[...])
    xu0, xu1 = _combine([au0_ref[...], au1_ref[...], au2_ref[...],
                         au3_ref[...]], cu_ref[...], biu_ref[...],
                        ru0_ref[...], ru1_ref[...])
    wli = wli_ref[...]
    wru = wru_ref[...]
    if full:
        wlu = wlu_ref[...]
        wri = wri_ref[...]
        _store_quarters(out_refs[0:4], _mm_half(xu0, xu1, wlu, 0),
                        _mm_half(xu0, xu1, wlu, 1))
        _store_quarters(out_refs[4:8], _mm_half(xi0, xi1, wli, 0),
                        _mm_half(xi0, xi1, wli, 1))
        out_refs[8][...] = _mm_half(xu0, xu1, wru, 0)
        out_refs[9][...] = _mm_half(xu0, xu1, wru, 1)
        out_refs[10][...] = _mm_half(xi0, xi1, wri, 0)
        out_refs[11][...] = _mm_half(xi0, xi1, wri, 1)
    else:
        _store_quarters(out_refs[0:4], _mm_half(xi0, xi1, wli, 0),
                        _mm_half(xi0, xi1, wli, 1))
        out_refs[4][...] = _mm_half(xu0, xu1, wru, 0)
        out_refs[5][...] = _mm_half(xu0, xu1, wru, 1)


def _kc_body(au0_ref, au1_ref, au2_ref, au3_ref, ru0_ref, ru1_ref, cu_ref,
             biu_ref, wh_ref, bh_ref, out_ref):
    xu0, xu1 = _combine([au0_ref[...], au1_ref[...], au2_ref[...],
                         au3_ref[...]], cu_ref[...], biu_ref[...],
                        ru0_ref[...], ru1_ref[...])
    wh = wh_ref[...]
    out_ref[...] = (_dotT(xu0, wh[:, 0:HH]) + _dotT(xu1, wh[:, HH:H])
                    + bh_ref[...])


def _rows(c):
    return pl.BlockSpec((BLK, c), lambda i: (i, 0))


def _whole(shape):
    return pl.BlockSpec(shape, lambda i: tuple(0 for _ in shape))


_q_out = [jax.ShapeDtypeStruct((N, HQ), _f32)]
_h_out = [jax.ShapeDtypeStruct((N, HH), _f32)]


def _ka_call(xu, xi, weu, wei, wlu, wru, wli, wri):
    return pl.pallas_call(
        _ka_body,
        grid=(GRID,),
        in_specs=[_rows(H), _rows(H)] + [_whole((H, H))] * 6,
        out_specs=[_rows(HQ)] * 8 + [_rows(HH)] * 4,
        out_shape=_q_out * 8 + _h_out * 4,
    )(xu, xi, weu, wei, wlu, wru, wli, wri)


def _kb_call(full, aggs, ri0, ri1, ru0, ru1, ci, cu, bui, biu,
             wlu, wru, wli, wri):
    n_q = 8 if full else 4
    n_h = 4 if full else 2
    return pl.pallas_call(
        functools.partial(_kb_body, full),
        grid=(GRID,),
        in_specs=[_rows(HQ)] * 8 + [_rows(HH)] * 4 + [_rows(HQ)] * 2
                 + [_whole((1, H))] * 2 + [_whole((H, H))] * 4,
        out_specs=[_rows(HQ)] * n_q + [_rows(HH)] * n_h,
        out_shape=_q_out * n_q + _h_out * n_h,
    )(*aggs, ri0, ri1, ru0, ru1, ci, cu, bui, biu, wlu, wru, wli, wri)


def _kc_call(auq, ru0, ru1, cu, biu, wh, bh):
    d_out = wh.shape[0]
    return pl.pallas_call(
        _kc_body,
        grid=(GRID,),
        in_specs=[_rows(HQ)] * 4 + [_rows(HH)] * 2
                 + [_rows(HQ), _whole((1, H)), _whole((d_out, H)),
                    _whole((1, d_out))],
        out_specs=_rows(d_out),
        out_shape=jax.ShapeDtypeStruct((N, d_out), _f32),
    )(*auq, ru0, ru1, cu, biu, wh, bh)


def _prep_edges(edge_index):
    src = edge_index[0].astype(jnp.int32)
    dst = edge_index[1].astype(jnp.int32)
    pad = EP - E
    src = jnp.concatenate([src, jnp.zeros((pad,), jnp.int32)])
    dummy = N + (jnp.arange(pad, dtype=jnp.int32) % (ACC - N))
    dst = jnp.concatenate([dst, dummy])
    return src.reshape(NS, NCH, CH), dst.reshape(NS, NCH, CH)


def kernel(x_user, x_item, edge_index_ui, edge_index_iu, W_enc_u, W_enc_i,
           Wl_ui, bl_ui, Wr_ui, Wl_iu, bl_iu, Wr_iu, Wh, bh):
    src_ui, dst_ui = _prep_edges(edge_index_ui)
    src_iu, dst_iu = _prep_edges(edge_index_iu)
    z2 = jnp.zeros((WBR, HQ), _f32)
    o2 = jnp.ones((WBR, HQ), _f32)

    bui = [bl_ui[l].reshape(1, H) for l in range(3)]
    biu = [bl_iu[l].reshape(1, H) for l in range(3)]
    bh2 = bh.reshape(1, -1)

    (yu0, yu1, yu2, yu3, yi0, yi1, yi2, yi3, ru0, ru1, ri0, ri1) = _ka_call(
        x_user, x_item, W_enc_u, W_enc_i, Wl_ui[0], Wr_iu[0], Wl_iu[0],
        Wr_ui[0])

    (ai0, ai1, ai2, ai3, au0, au1, au2, au3, ci, cu) = _sc_agg2_cnt(
        yu0, yu1, yu2, yu3, yi0, yi1, yi2, yi3,
        src_ui, dst_ui, src_iu, dst_iu, z2, o2)

    (yu0, yu1, yu2, yu3, yi0, yi1, yi2, yi3, ru0, ru1, ri0, ri1) = _kb_call(
        True, (ai0, ai1, ai2, ai3, au0, au1, au2, au3), ri0, ri1, ru0, ru1,
        ci, cu, bui[0], biu[0], Wl_ui[1], Wr_iu[1], Wl_iu[1], Wr_ui[1])

    (ai0, ai1, ai2, ai3, au0, au1, au2, au3) = _sc_agg2(
        yu0, yu1, yu2, yu3, yi0, yi1, yi2, yi3,
        src_ui, dst_ui, src_iu, dst_iu, z2)

    (yi0, yi1, yi2, yi3, ru0, ru1) = _kb_call(
        False, (ai0, ai1, ai2, ai3, au0, au1, au2, au3), ri0, ri1, ru0, ru1,
        ci, cu, bui[1], biu[1], Wl_ui[2], Wr_iu[2], Wl_iu[2], Wr_ui[2])

    au0, au1, au2, au3 = _sc_agg1(yi0, yi1, yi2, yi3, src_iu, dst_iu, z2)

    return _kc_call((au0, au1, au2, au3), ru0, ru1, cu, biu[2], Wh, bh2)
